# baseline (speedup 1.0000x reference)
import functools

import jax
import jax.numpy as jnp
from jax import lax
from jax.experimental import pallas as pl
from jax.experimental.pallas import tpu as pltpu
from jax.experimental.pallas import tpu_sc as plsc

N, F, D = 524288, 26, 4
NF = N * F
NW = 32
PER_W = NF // NW
CHUNK = 2048
STEPS = PER_W // CHUNK


def _make_sc_kernel():
    mesh = plsc.VectorSubcoreMesh(core_axis_name="c", subcore_axis_name="s")

    @functools.partial(
        pl.kernel, mesh=mesh,
        out_type=jax.ShapeDtypeStruct((NF * D,), jnp.float32),
        scratch_types=[
            pltpu.VMEM((CHUNK,), jnp.float32),
            pltpu.VMEM((CHUNK,), jnp.int32),
            pltpu.VMEM((D * CHUNK,), jnp.float32),
            pltpu.VMEM((16,), jnp.int32),
            pltpu.VMEM((16,), jnp.int32),
        ],
    )
    def _sc_lvr(x_hbm, i_hbm, kv_hbm, gv_hbm, o_hbm, xv, iv, ov, kvv, gvv):
        wid = lax.axis_index("s") * 2 + lax.axis_index("c")
        base = wid * PER_W
        pltpu.sync_copy(kv_hbm, kvv)
        pltpu.sync_copy(gv_hbm, gvv)

        def step(t, carry):
            off = base + t * CHUNK
            pltpu.sync_copy(x_hbm.at[pl.ds(off, CHUNK)], xv)
            pltpu.sync_copy(i_hbm.at[pl.ds(off, CHUNK)], iv)
            kvec = kvv[pl.ds(0, 16)]
            group = gvv[pl.ds(0, 16)]

            def inner(j, c2):
                xe = xv[pl.ds(j * 16, 16)]
                ie = iv[pl.ds(j * 16, 16)]
                for m in range(4):
                    enc = jnp.zeros((16,), jnp.float32)
                    for g in range(4):
                        xs = xe[m * 4 + g]
                        isc = ie[m * 4 + g]
                        e_g = jnp.where(kvec < isc, 1.0,
                                        jnp.where(kvec > isc, 0.0, xs))
                        enc = jnp.where(group == g, e_g, enc)
                    ov[pl.ds(j * 64 + m * 16, 16)] = enc
                return c2

            lax.fori_loop(0, CHUNK // 16, inner, 0)
            pltpu.sync_copy(ov, o_hbm.at[pl.ds(off * D, CHUNK * D)])
            return carry

        lax.fori_loop(0, STEPS, step, 0)

    return _sc_lvr


_SC_LVR = _make_sc_kernel()


def kernel(x, indices):
    kvec = jnp.arange(16, dtype=jnp.int32) % D
    group = jnp.arange(16, dtype=jnp.int32) // D
    out = _SC_LVR(x.reshape(NF), indices.reshape(NF), kvec, group)
    return out.reshape(N, F, D)


# speedup vs baseline: 72.4255x; 72.4255x over previous
import jax
import jax.numpy as jnp
from jax.experimental import pallas as pl
from jax.experimental.pallas import tpu as pltpu

N, F, D = 524288, 26, 4
LANES = 128
NT = N // LANES
BNT = 256
BN = BNT * LANES


CH = 8


def _lvr_block(x_ref, idx_ref, o_ref):
    def body(c, carry):
        sl = pl.ds(c * CH * LANES, CH * LANES)
        x3 = x_ref[:, sl].reshape(F, CH, LANES)
        i3 = idx_ref[:, sl].reshape(F, CH, LANES)
        base = c * CH * D
        for k in range(D):
            if k == 0:
                plane = jnp.where(i3 > 0, 1.0, x3)
            elif k == D - 1:
                plane = jnp.where(i3 < D - 1, 0.0, x3)
            else:
                plane = jnp.where(i3 > k, 1.0, jnp.where(i3 < k, 0.0, x3))
            o_ref[:, pl.Slice(base + k, CH, D), :] = plane
        return carry
    jax.lax.fori_loop(0, BNT // CH, body, 0)


def kernel(x, indices):
    out = pl.pallas_call(
        _lvr_block,
        grid=(NT // BNT,),
        in_specs=[
            pl.BlockSpec((F, BN), lambda i: (0, i)),
            pl.BlockSpec((F, BN), lambda i: (0, i)),
        ],
        out_specs=pl.BlockSpec((F, D * BNT, LANES), lambda i: (0, i, 0)),
        out_shape=jax.ShapeDtypeStruct((F, D * NT, LANES), jnp.float32),
        compiler_params=pltpu.CompilerParams(
            dimension_semantics=("parallel",)),
    )(x.T, indices.T)
    return out.reshape(F, NT, D, LANES).transpose(1, 3, 0, 2).reshape(N, F, D)
